# baseline (speedup 1.0000x reference)
import functools

import jax
import jax.numpy as jnp
from jax import lax
from jax.experimental import pallas as pl
from jax.experimental.pallas import tpu as pltpu
from jax.experimental.pallas import tpu_sc as plsc

_NC, _NS = 2, 16
_NW = _NC * _NS
_CHUNK = 2048


def kernel(memory, input_logits):
    size, dim = memory.shape
    num = input_logits.shape[0]
    assert num % _CHUNK == 0
    lchunks = num // _CHUNK
    nfull = size // _CHUNK
    tail = size - nfull * _CHUNK
    niter = (nfull + _NW - 1) // _NW
    memt = memory.T
    logt = input_logits.T

    mesh = plsc.VectorSubcoreMesh(core_axis_name="c", subcore_axis_name="s")

    @functools.partial(
        pl.kernel,
        out_type=jax.ShapeDtypeStruct((dim, size), memory.dtype),
        mesh=mesh,
        compiler_params=pltpu.CompilerParams(use_tc_tiling_on_sc=True),
        scratch_types=[
            pltpu.VMEM((dim, _CHUNK), jnp.float32),
        ],
    )
    def run(mem_hbm, log_hbm, out_hbm, buf):
        w = lax.axis_index("s") * _NC + lax.axis_index("c")

        def body(j, carry):
            c = w + _NW * j
            start = c * _CHUNK

            @pl.when(c < lchunks)
            def _():
                pltpu.sync_copy(log_hbm.at[:, pl.ds(start, _CHUNK)], buf)
                pltpu.sync_copy(buf, out_hbm.at[:, pl.ds(start, _CHUNK)])

            @pl.when(jnp.logical_and(c >= lchunks, c < nfull))
            def _():
                pltpu.sync_copy(mem_hbm.at[:, pl.ds(start, _CHUNK)], buf)
                pltpu.sync_copy(buf, out_hbm.at[:, pl.ds(start, _CHUNK)])

            return carry

        lax.fori_loop(0, niter, body, 0)

    outt = run(memt, logt)

    tail_block = 128
    nblocks = pl.cdiv(tail, tail_block)
    first = (nfull * _CHUNK) // tail_block

    def tc_body(mem_ref, alias_ref, out_ref, idx_ref):
        del alias_ref
        out_ref[...] = mem_ref[...]
        idx_ref[0] = jnp.int32(num % size)

    outt, new_index = pl.pallas_call(
        tc_body,
        grid=(nblocks,),
        in_specs=[
            pl.BlockSpec((dim, tail_block), lambda i: (0, first + i)),
            pl.BlockSpec(memory_space=pl.ANY),
        ],
        out_specs=[
            pl.BlockSpec((dim, tail_block), lambda i: (0, first + i)),
            pl.BlockSpec(memory_space=pltpu.SMEM),
        ],
        out_shape=[
            jax.ShapeDtypeStruct((dim, size), memory.dtype),
            jax.ShapeDtypeStruct((1,), jnp.int32),
        ],
        input_output_aliases={1: 0},
    )(memt, outt)
    return (outt.T, new_index[0])


# speedup vs baseline: 1.0357x; 1.0357x over previous
import functools

import jax
import jax.numpy as jnp
from jax import lax
from jax.experimental import pallas as pl
from jax.experimental.pallas import tpu as pltpu
from jax.experimental.pallas import tpu_sc as plsc

_NC, _NS = 2, 16
_NW = _NC * _NS
_CHUNK = 1024


def kernel(memory, input_logits):
    size, dim = memory.shape
    num = input_logits.shape[0]
    assert num % _CHUNK == 0
    lchunks = num // _CHUNK
    nfull = size // _CHUNK
    tail = size - nfull * _CHUNK
    npairs = (pl.cdiv(nfull, _NW) + 1) // 2
    memt = memory.T
    logt = input_logits.T

    mesh = plsc.VectorSubcoreMesh(core_axis_name="c", subcore_axis_name="s")

    @functools.partial(
        pl.kernel,
        out_type=jax.ShapeDtypeStruct((dim, size), memory.dtype),
        mesh=mesh,
        compiler_params=pltpu.CompilerParams(use_tc_tiling_on_sc=True),
        scratch_types=[
            pltpu.VMEM((dim, _CHUNK), jnp.float32),
            pltpu.VMEM((dim, _CHUNK), jnp.float32),
            pltpu.SemaphoreType.DMA((4,)),
        ],
    )
    def run(mem_hbm, log_hbm, out_hbm, buf0, buf1, sems):
        w = lax.axis_index("s") * _NC + lax.axis_index("c")

        def start_read(c, buf, sem):
            start = c * _CHUNK

            @pl.when(c < lchunks)
            def _():
                pltpu.async_copy(log_hbm.at[:, pl.ds(start, _CHUNK)], buf, sem)

            @pl.when(jnp.logical_and(c >= lchunks, c < nfull))
            def _():
                pltpu.async_copy(mem_hbm.at[:, pl.ds(start, _CHUNK)], buf, sem)

        def wait_read(c, buf, sem):
            @pl.when(c < nfull)
            def _():
                pltpu.make_async_copy(
                    mem_hbm.at[:, pl.ds(c * _CHUNK, _CHUNK)], buf, sem).wait()

        def write(c, buf, sem):
            @pl.when(c < nfull)
            def _():
                pltpu.async_copy(buf, out_hbm.at[:, pl.ds(c * _CHUNK, _CHUNK)],
                                 sem)

        def wait_write(c, buf, sem):
            @pl.when(c < nfull)
            def _():
                pltpu.make_async_copy(
                    buf, out_hbm.at[:, pl.ds(c * _CHUNK, _CHUNK)], sem).wait()

        def body(t, carry):
            c0 = w + (2 * t) * _NW
            c1 = w + (2 * t + 1) * _NW
            start_read(c0, buf0, sems.at[0])
            start_read(c1, buf1, sems.at[1])
            wait_read(c0, buf0, sems.at[0])
            write(c0, buf0, sems.at[2])
            wait_read(c1, buf1, sems.at[1])
            write(c1, buf1, sems.at[3])
            wait_write(c0, buf0, sems.at[2])
            wait_write(c1, buf1, sems.at[3])
            return carry

        lax.fori_loop(0, npairs, body, 0)

    outt = run(memt, logt)

    tail_block = 128
    nblocks = pl.cdiv(tail, tail_block)
    first = (nfull * _CHUNK) // tail_block

    def tc_body(mem_ref, alias_ref, out_ref, idx_ref):
        del alias_ref
        out_ref[...] = mem_ref[...]
        idx_ref[0] = jnp.int32(num % size)

    outt, new_index = pl.pallas_call(
        tc_body,
        grid=(nblocks,),
        in_specs=[
            pl.BlockSpec((dim, tail_block), lambda i: (0, first + i)),
            pl.BlockSpec(memory_space=pl.ANY),
        ],
        out_specs=[
            pl.BlockSpec((dim, tail_block), lambda i: (0, first + i)),
            pl.BlockSpec(memory_space=pltpu.SMEM),
        ],
        out_shape=[
            jax.ShapeDtypeStruct((dim, size), memory.dtype),
            jax.ShapeDtypeStruct((1,), jnp.int32),
        ],
        input_output_aliases={1: 0},
    )(memt, outt)
    return (outt.T, new_index[0])


# speedup vs baseline: 1.5413x; 1.4881x over previous
import jax
import jax.numpy as jnp
from jax.experimental import pallas as pl
from jax.experimental.pallas import tpu as pltpu

_BLOCK = 98304


def kernel(memory, input_logits):
    size, dim = memory.shape
    num = input_logits.shape[0]
    assert num <= _BLOCK
    memt = memory.T
    logt = input_logits.T
    grid = (pl.cdiv(size, _BLOCK),)

    def body(mem_ref, logits_ref, out_ref, idx_ref):
        i = pl.program_id(0)

        @pl.when(i == 0)
        def _():
            out_ref[:, 0:num] = logits_ref[...]
            out_ref[:, num:_BLOCK] = mem_ref[:, num:_BLOCK]
            idx_ref[0] = jnp.int32(num % size)

        @pl.when(i > 0)
        def _():
            out_ref[...] = mem_ref[...]

    outt, new_index = pl.pallas_call(
        body,
        grid=grid,
        in_specs=[
            pl.BlockSpec((dim, _BLOCK), lambda i: (0, i)),
            pl.BlockSpec((dim, num), lambda i: (0, 0)),
        ],
        out_specs=[
            pl.BlockSpec((dim, _BLOCK), lambda i: (0, i)),
            pl.BlockSpec(memory_space=pltpu.SMEM),
        ],
        out_shape=[
            jax.ShapeDtypeStruct((dim, size), memory.dtype),
            jax.ShapeDtypeStruct((1,), jnp.int32),
        ],
    )(memt, logt)
    return (outt.T, new_index[0])


# speedup vs baseline: 1.5443x; 1.0020x over previous
import jax
import jax.numpy as jnp
from jax.experimental import pallas as pl
from jax.experimental.pallas import tpu as pltpu

_BLOCK = 106496


def kernel(memory, input_logits):
    size, dim = memory.shape
    num = input_logits.shape[0]
    assert num <= _BLOCK
    memt = memory.T
    logt = input_logits.T
    grid = (pl.cdiv(size, _BLOCK),)

    def body(mem_ref, logits_ref, out_ref, idx_ref):
        i = pl.program_id(0)

        @pl.when(i == 0)
        def _():
            out_ref[:, 0:num] = logits_ref[...]
            out_ref[:, num:_BLOCK] = mem_ref[:, num:_BLOCK]
            idx_ref[0] = jnp.int32(num % size)

        @pl.when(i > 0)
        def _():
            out_ref[...] = mem_ref[...]

    outt, new_index = pl.pallas_call(
        body,
        grid=grid,
        in_specs=[
            pl.BlockSpec((dim, _BLOCK), lambda i: (0, i)),
            pl.BlockSpec((dim, num), lambda i: (0, 0)),
        ],
        out_specs=[
            pl.BlockSpec((dim, _BLOCK), lambda i: (0, i)),
            pl.BlockSpec(memory_space=pltpu.SMEM),
        ],
        out_shape=[
            jax.ShapeDtypeStruct((dim, size), memory.dtype),
            jax.ShapeDtypeStruct((1,), jnp.int32),
        ],
    )(memt, logt)
    return (outt.T, new_index[0])
